# baseline (speedup 1.0000x reference)
import jax
import jax.numpy as jnp
from jax import lax
from jax.experimental import pallas as pl
from jax.experimental.pallas import tpu as pltpu
from jax.experimental.pallas import tpu_sc as plsc

_B = 64
_PERIODS = 128
_SEG_A = 48
_SEG_B = 80
_NW = 32
_CHUNK_T = 16
_CHUNKS = 3 * (_B // 8) * (_PERIODS // _CHUNK_T)
_CHUNKS_PER_W = _CHUNKS // _NW


def _sc_body(x_hbm, o_hbm, buf0, buf1, buf2, buf3, buf4, buf5, outv,
             sem0, sem1, sem2, sem3, sem4, sem5, osem):
    cid = lax.axis_index("c")
    sid = lax.axis_index("s")
    w = sid * 2 + cid

    iota = lax.iota(jnp.int32, 16)
    inv = jnp.where(iota % 2 == 0, jnp.float32(1.0 / _SEG_A),
                    jnp.float32(1.0 / _SEG_B))

    bufs = (buf0, buf1, buf2, buf3, buf4, buf5)
    sems = (sem0, sem1, sem2, sem3, sem4, sem5)

    def chunk_coords(j):
        kg = w * _CHUNKS_PER_W + j
        c = kg // 64
        rem = kg - c * 64
        tr = rem // 8
        cb = rem - tr * 8
        return c, tr, cb

    def chunk_slice(j):
        c, tr, cb = chunk_coords(j)
        return x_hbm.at[c, tr, pl.ds(cb * _CHUNK_T, _CHUNK_T)]

    xor_idx = {d: iota ^ d for d in (8, 4, 2, 1)}
    sel_lo = {d: (iota & d) == 0 for d in (8, 4, 2, 1)}

    def merge(a, b, d):
        ta = a + a.at[xor_idx[d]].get(mode="promise_in_bounds")
        tb = b + b.at[xor_idx[d]].get(mode="promise_in_bounds")
        return jnp.where(sel_lo[d], ta, tb)

    def compute(buf, j):
        def row(r, carry):
            for half in range(2):
                vs = {}
                for tl in range(8):
                    t = half * 8 + tl
                    vs[2 * tl] = (buf[t, r, pl.ds(0, 16)]
                                  + buf[t, r, pl.ds(16, 16)]
                                  + buf[t, r, pl.ds(32, 16)])
                    vs[2 * tl + 1] = (buf[t, r, pl.ds(48, 16)]
                                      + buf[t, r, pl.ds(64, 16)]
                                      + buf[t, r, pl.ds(80, 16)]
                                      + buf[t, r, pl.ds(96, 16)]
                                      + buf[t, r, pl.ds(112, 16)])
                for d in (8, 4, 2, 1):
                    vs = {i: merge(vs[i], vs[i | d], d)
                          for i in vs if i & d == 0}
                outv[j, r, pl.ds(16 * half, 16)] = vs[0] * inv
            return carry
        lax.fori_loop(0, 8, row, 0)

    copies = [pltpu.make_async_copy(chunk_slice(j), bufs[j], sems[j])
              for j in range(_CHUNKS_PER_W)]
    for cp in copies:
        cp.start()
    for j in range(_CHUNKS_PER_W):
        copies[j].wait()
        compute(bufs[j], j)

    fired = []
    for j in range(_CHUNKS_PER_W):
        c, tr, cb = chunk_coords(j)
        obase = (c * 16 + tr * 2 + cb // 4) * 1024 + (cb % 4) * 32
        for r in range(8):
            cp = pltpu.make_async_copy(
                outv.at[j, r], o_hbm.at[pl.ds(obase + r * 128, 32)], osem)
            cp.start()
            fired.append(cp)
    for cp in fired:
        cp.wait()


def kernel(inputs):
    x5 = jnp.transpose(inputs, (2, 0, 1)).reshape(3, 8, 8, 128, 128)
    x5 = jnp.transpose(x5, (0, 1, 3, 2, 4))
    mesh = plsc.VectorSubcoreMesh(core_axis_name="c", subcore_axis_name="s")
    run = pl.kernel(
        _sc_body,
        out_type=jax.ShapeDtypeStruct((48 * 1024,), jnp.float32),
        mesh=mesh,
        scratch_types=(
            [pltpu.VMEM((_CHUNK_T, 8, 128), jnp.float32)] * _CHUNKS_PER_W
            + [pltpu.VMEM((_CHUNKS_PER_W, 8, 32), jnp.float32)]
            + [pltpu.SemaphoreType.DMA] * (_CHUNKS_PER_W + 1)
        ),
        compiler_params=pltpu.CompilerParams(needs_layout_passes=False),
    )
    out = run(x5)
    out = out.reshape(3, _B // 8, 2, 8, 128)
    out = out.transpose(1, 3, 2, 4, 0)
    return out.reshape(_B, 2 * _PERIODS, 3)


# speedup vs baseline: 1.0806x; 1.0806x over previous
import jax
import jax.numpy as jnp
from jax import lax
from jax.experimental import pallas as pl
from jax.experimental.pallas import tpu as pltpu
from jax.experimental.pallas import tpu_sc as plsc

_B = 64
_PERIODS = 128
_SEG_A = 48
_SEG_B = 80
_NW = 32
_CHUNK_T = 16
_CHUNKS = 3 * (_B // 8) * (_PERIODS // _CHUNK_T)
_CHUNKS_PER_W = _CHUNKS // _NW


def _sc_body(x_hbm, o_hbm, buf0, buf1, buf2, outv, sem0, sem1, sem2, osem):
    cid = lax.axis_index("c")
    sid = lax.axis_index("s")
    w = sid * 2 + cid

    iota = lax.iota(jnp.int32, 16)
    inv = jnp.where(iota % 2 == 0, jnp.float32(1.0 / _SEG_A),
                    jnp.float32(1.0 / _SEG_B))

    bufs = (buf0, buf1, buf2)
    sems = (sem0, sem1, sem2)

    def chunk_coords(j):
        kg = w * _CHUNKS_PER_W + j
        c = kg // 64
        rem = kg - c * 64
        tr = rem // 8
        cb = rem - tr * 8
        return c, tr, cb

    def chunk_slice(j):
        c, tr, cb = chunk_coords(j)
        return x_hbm.at[c, tr, pl.ds(cb * _CHUNK_T, _CHUNK_T)]

    xor_idx = {d: iota ^ d for d in (8, 4, 2, 1)}
    sel_lo = {d: (iota & d) == 0 for d in (8, 4, 2, 1)}

    def merge(a, b, d):
        ta = a + a.at[xor_idx[d]].get(mode="promise_in_bounds")
        tb = b + b.at[xor_idx[d]].get(mode="promise_in_bounds")
        return jnp.where(sel_lo[d], ta, tb)

    def compute(buf, j):
        def row(r, carry):
            for half in range(2):
                vs = {}
                for tl in range(8):
                    t = half * 8 + tl
                    vs[2 * tl] = (buf[t, r, pl.ds(0, 16)]
                                  + buf[t, r, pl.ds(16, 16)]
                                  + buf[t, r, pl.ds(32, 16)])
                    vs[2 * tl + 1] = (buf[t, r, pl.ds(48, 16)]
                                      + buf[t, r, pl.ds(64, 16)]
                                      + buf[t, r, pl.ds(80, 16)]
                                      + buf[t, r, pl.ds(96, 16)]
                                      + buf[t, r, pl.ds(112, 16)])
                for d in (8, 4, 2, 1):
                    vs = {i: merge(vs[i], vs[i | d], d)
                          for i in vs if i & d == 0}
                outv[j, r, pl.ds(16 * half, 16)] = vs[0] * inv
            return carry
        lax.fori_loop(0, 8, row, 0)

    for p in range(3):
        pltpu.make_async_copy(chunk_slice(p), bufs[p], sems[p]).start()

    def tbody(t, carry):
        for p in range(3):
            j = 3 * t + p
            pltpu.make_async_copy(chunk_slice(j), bufs[p], sems[p]).wait()
            compute(bufs[p], j)

            @pl.when(j + 3 < _CHUNKS_PER_W)
            def _():
                pltpu.make_async_copy(
                    chunk_slice(j + 3), bufs[p], sems[p]).start()
        return carry

    lax.fori_loop(0, _CHUNKS_PER_W // 3, tbody, 0)

    fired = []
    for j in range(_CHUNKS_PER_W):
        c, tr, cb = chunk_coords(j)
        obase = (c * 16 + tr * 2 + cb // 4) * 1024 + (cb % 4) * 32
        for r in range(8):
            cp = pltpu.make_async_copy(
                outv.at[j, r], o_hbm.at[pl.ds(obase + r * 128, 32)], osem)
            cp.start()
            fired.append(cp)
    for cp in fired:
        cp.wait()


def kernel(inputs):
    x5 = jnp.transpose(inputs, (2, 0, 1)).reshape(3, 8, 8, 128, 128)
    x5 = jnp.transpose(x5, (0, 1, 3, 2, 4))
    mesh = plsc.VectorSubcoreMesh(core_axis_name="c", subcore_axis_name="s")
    run = pl.kernel(
        _sc_body,
        out_type=jax.ShapeDtypeStruct((48 * 1024,), jnp.float32),
        mesh=mesh,
        scratch_types=(
            [pltpu.VMEM((_CHUNK_T, 8, 128), jnp.float32)] * 3
            + [pltpu.VMEM((_CHUNKS_PER_W, 8, 32), jnp.float32)]
            + [pltpu.SemaphoreType.DMA] * 4
        ),
        compiler_params=pltpu.CompilerParams(needs_layout_passes=False),
    )
    out = run(x5)
    out = out.reshape(3, _B // 8, 2, 8, 128)
    out = out.transpose(1, 3, 2, 4, 0)
    return out.reshape(_B, 2 * _PERIODS, 3)


# speedup vs baseline: 1.0986x; 1.0167x over previous
import jax
import jax.numpy as jnp
from jax import lax
from jax.experimental import pallas as pl
from jax.experimental.pallas import tpu as pltpu
from jax.experimental.pallas import tpu_sc as plsc

_B = 64
_PERIODS = 128
_SEG_A = 48
_SEG_B = 80
_NW = 32
_CHUNK_T = 16
_CHUNKS = 3 * (_B // 8) * (_PERIODS // _CHUNK_T)
_CHUNKS_PER_W = _CHUNKS // _NW


def _sc_body(x_hbm, o_hbm, buf0, buf1, buf2, outv, sem0, sem1, sem2, osem):
    cid = lax.axis_index("c")
    sid = lax.axis_index("s")
    w = sid * 2 + cid

    iota = lax.iota(jnp.int32, 16)
    inv = jnp.where(iota % 2 == 0, jnp.float32(1.0 / _SEG_A),
                    jnp.float32(1.0 / _SEG_B))

    bufs = (buf0, buf1, buf2)
    sems = (sem0, sem1, sem2)

    def chunk_coords(j):
        kg = w * _CHUNKS_PER_W + j
        c = kg // 64
        rem = kg - c * 64
        tr = rem // 8
        cb = rem - tr * 8
        return c, tr, cb

    def chunk_slice(j):
        c, tr, cb = chunk_coords(j)
        return x_hbm.at[c, tr, pl.ds(cb * _CHUNK_T, _CHUNK_T)]

    xor_idx = {d: iota ^ d for d in (8, 4, 2, 1)}
    sel_lo = {d: (iota & d) == 0 for d in (8, 4, 2, 1)}

    def merge(a, b, d):
        ta = a + a.at[xor_idx[d]].get(mode="promise_in_bounds")
        tb = b + b.at[xor_idx[d]].get(mode="promise_in_bounds")
        return jnp.where(sel_lo[d], ta, tb)

    def compute(buf, j):
        def row(r, carry):
            for half in range(2):
                vs = {}
                for tl in range(8):
                    t = half * 8 + tl
                    vs[2 * tl] = (buf[t, r, pl.ds(0, 16)]
                                  + buf[t, r, pl.ds(16, 16)]
                                  + buf[t, r, pl.ds(32, 16)])
                    vs[2 * tl + 1] = (buf[t, r, pl.ds(48, 16)]
                                      + buf[t, r, pl.ds(64, 16)]
                                      + buf[t, r, pl.ds(80, 16)]
                                      + buf[t, r, pl.ds(96, 16)]
                                      + buf[t, r, pl.ds(112, 16)])
                for d in (8, 4, 2, 1):
                    vs = {i: merge(vs[i], vs[i | d], d)
                          for i in vs if i & d == 0}
                outv[j, r, pl.ds(16 * half, 16)] = vs[0] * inv
            return carry
        lax.fori_loop(0, 8, row, 0)

    for p in range(3):
        pltpu.make_async_copy(chunk_slice(p), bufs[p], sems[p]).start()

    def out_copy(j, r):
        c, tr, cb = chunk_coords(j)
        obase = (c * 16 + tr * 2 + cb // 4) * 1024 + (cb % 4) * 32
        return pltpu.make_async_copy(
            outv.at[j, r], o_hbm.at[pl.ds(obase + r * 128, 32)], osem)

    def tbody(t, carry):
        for p in range(3):
            j = 3 * t + p
            pltpu.make_async_copy(chunk_slice(j), bufs[p], sems[p]).wait()
            compute(bufs[p], j)

            @pl.when(j + 3 < _CHUNKS_PER_W)
            def _():
                pltpu.make_async_copy(
                    chunk_slice(j + 3), bufs[p], sems[p]).start()

            for r in range(8):
                out_copy(j, r).start()
        return carry

    lax.fori_loop(0, _CHUNKS_PER_W // 3, tbody, 0)

    for j in range(_CHUNKS_PER_W):
        for r in range(8):
            out_copy(j, r).wait()


def kernel(inputs):
    x5 = jnp.transpose(inputs, (2, 0, 1)).reshape(3, 8, 8, 128, 128)
    x5 = jnp.transpose(x5, (0, 1, 3, 2, 4))
    mesh = plsc.VectorSubcoreMesh(core_axis_name="c", subcore_axis_name="s")
    run = pl.kernel(
        _sc_body,
        out_type=jax.ShapeDtypeStruct((48 * 1024,), jnp.float32),
        mesh=mesh,
        scratch_types=(
            [pltpu.VMEM((_CHUNK_T, 8, 128), jnp.float32)] * 3
            + [pltpu.VMEM((_CHUNKS_PER_W, 8, 32), jnp.float32)]
            + [pltpu.SemaphoreType.DMA] * 4
        ),
        compiler_params=pltpu.CompilerParams(needs_layout_passes=False),
    )
    out = run(x5)
    out = out.reshape(3, _B // 8, 2, 8, 128)
    out = out.transpose(1, 3, 2, 4, 0)
    return out.reshape(_B, 2 * _PERIODS, 3)
